# baseline (speedup 1.0000x reference)
import functools

import jax
import jax.numpy as jnp
from jax import lax
from jax.experimental import pallas as pl
from jax.experimental.pallas import tpu as pltpu
from jax.experimental.pallas import tpu_sc as plsc

N = 4096
E = 65536
D = 128
D_EDGE = 16
NSIG = 15
_SIGMAS = [1.5 ** x for x in range(15)]
SLOPE = 0.01
SKIP_H = 0.5
X_CONN = 0.25
PW = 144

NC = 2
NS = 16
NW = NC * NS
EPT = (2 * E) // NW
GCH = 128
RCH = 512


def _leaky(x):
    return jnp.where(x >= 0, x, SLOPE * x)


def _proj_body(h_ref, w1a_ref, w1b_ref, wq_ref, wk_ref, wv_ref,
               ts_ref, td_ref, q_ref, kk_ref, vv_ref):
    h = h_ref[...]
    mm = lambda x, wr: lax.dot_general(x, wr[0], (((1,), (1,)), ((), ())),
                                       preferred_element_type=jnp.float32)
    ts_ref[...] = mm(h, w1a_ref)
    td_ref[...] = mm(h, w1b_ref)
    q_ref[...] = _leaky(mm(h, wq_ref))
    kk_ref[...] = _leaky(mm(h, wk_ref))
    vv_ref[...] = mm(h, wv_ref)


def _run_proj(h2, w1a2, w1b2, wq2, wk2, wv2):
    blk = 1024
    nblk = (2 * N) // blk
    half = nblk // 2
    wspec = pl.BlockSpec((1, D, D), lambda i: (i // half, 0, 0))
    nspec = pl.BlockSpec((blk, D), lambda i: (i, 0))
    return pl.pallas_call(
        _proj_body,
        grid=(nblk,),
        in_specs=[nspec, wspec, wspec, wspec, wspec, wspec],
        out_specs=[
            nspec, nspec, nspec,
            pl.BlockSpec((blk, D), lambda i: ((1 - i // half) * half + i % half, 0)),
            pl.BlockSpec((blk, D), lambda i: ((1 - i // half) * half + i % half, 0)),
        ],
        out_shape=[jax.ShapeDtypeStruct((2 * N, D), jnp.float32)] * 5,
    )(h2, w1a2, w1b2, wq2, wk2, wv2)


def _gather_body(ts_hbm, td_hbm, src_hbm, dst_hbm, os_hbm, od_hbm,
                 idxs_v, idxd_v, rows_v, sem):
    c = lax.axis_index("c")
    s = lax.axis_index("s")
    wid = c * NS + s
    ebase = wid * EPT

    pltpu.sync_copy(src_hbm.at[pl.ds(ebase, EPT)], idxs_v)
    pltpu.sync_copy(dst_hbm.at[pl.ds(ebase, EPT)], idxd_v)

    def one_table(tab_hbm, idx_v, out_hbm):
        def chunk(ch, _):
            rbase = ch * RCH
            cps = []
            for j in range(RCH // GCH):
                cps.append(pltpu.async_copy(
                    tab_hbm.at[idx_v.at[pl.ds(rbase + j * GCH, GCH)]],
                    rows_v.at[pl.ds(j * GCH, GCH)], sem))
            for cp in cps:
                cp.wait()
            pltpu.sync_copy(rows_v, out_hbm.at[pl.ds(ebase + rbase, RCH)])
            return ()

        lax.fori_loop(0, EPT // RCH, chunk, (), unroll=False)

    one_table(ts_hbm, idxs_v, os_hbm)
    one_table(td_hbm, idxd_v, od_hbm)


def _run_gather(tables_s, tables_d, src2, dst_g):
    mesh = plsc.VectorSubcoreMesh(core_axis_name="c", subcore_axis_name="s")
    k = pl.kernel(
        _gather_body,
        mesh=mesh,
        out_type=[
            jax.ShapeDtypeStruct((2 * E, D), jnp.float32),
            jax.ShapeDtypeStruct((2 * E, D), jnp.float32),
        ],
        scratch_types=[
            pltpu.VMEM((EPT,), jnp.int32),
            pltpu.VMEM((EPT,), jnp.int32),
            pltpu.VMEM((RCH, D), jnp.float32),
            pltpu.SemaphoreType.DMA,
        ],
        compiler_params=pltpu.CompilerParams(use_tc_tiling_on_sc=True),
    )
    return k(tables_s, tables_d, src2, dst_g)


def _cgather_body(ct_hbm, src_hbm, dst_hbm, ocs_hbm, ocd_hbm,
                  idx_v, rows_v, sem):
    c = lax.axis_index("c")
    s = lax.axis_index("s")
    wid = c * NS + s
    ebase = wid * EPT

    def one_table(eidx_hbm, out_hbm):
        pltpu.sync_copy(eidx_hbm.at[pl.ds(ebase, EPT)], idx_v)

        def chunk(ch, _):
            rbase = ch * RCH
            cps = []
            for j in range(RCH // GCH):
                cps.append(pltpu.async_copy(
                    ct_hbm.at[idx_v.at[pl.ds(rbase + j * GCH, GCH)]],
                    rows_v.at[pl.ds(j * GCH, GCH)], sem))
            for cp in cps:
                cp.wait()
            pltpu.sync_copy(rows_v, out_hbm.at[pl.ds(ebase + rbase, RCH)])
            return ()

        lax.fori_loop(0, EPT // RCH, chunk, (), unroll=False)

    one_table(src_hbm, ocs_hbm)
    one_table(dst_hbm, ocd_hbm)


def _run_cgather(coords16, src2, dst_g):
    mesh = plsc.VectorSubcoreMesh(core_axis_name="c", subcore_axis_name="s")
    k = pl.kernel(
        _cgather_body,
        mesh=mesh,
        out_type=[
            jax.ShapeDtypeStruct((2 * E, D_EDGE), jnp.float32),
            jax.ShapeDtypeStruct((2 * E, D_EDGE), jnp.float32),
        ],
        scratch_types=[
            pltpu.VMEM((EPT,), jnp.int32),
            pltpu.VMEM((RCH, D_EDGE), jnp.float32),
            pltpu.SemaphoreType.DMA,
        ],
        compiler_params=pltpu.CompilerParams(use_tc_tiling_on_sc=False),
    )
    return k(coords16, src2, dst_g)


def _edge_body(os_ref, od_ref, ef_ref, ocs_ref, ocd_ref, w1c_ref, w1d_ref,
               b1_ref, lng_ref, lnb_ref, w2_ref, b2_ref, wc1_ref, bc1_ref,
               wc2_ref, bc2_ref, pm_ref, px_ref):
    ksig = lax.broadcasted_iota(jnp.int32, (1, NSIG), 1).astype(jnp.float32)
    inv_sig = jnp.exp(-ksig * jnp.log(jnp.float32(1.5)))

    mm = lambda x, wr: lax.dot_general(x, wr[0], (((1,), (1,)), ((), ())),
                                       preferred_element_type=jnp.float32)
    xs = ocs_ref[...] - ocd_ref[...]
    sq = jnp.sum(xs * xs, axis=1, keepdims=True)
    rbf = jnp.exp(-sq * inv_sig)
    h1 = (os_ref[...] + od_ref[...] + mm(ef_ref[...], w1c_ref)
          + mm(rbf, w1d_ref) + b1_ref[0])
    h1 = _leaky(h1)
    m = jnp.mean(h1, axis=1, keepdims=True)
    cen = h1 - m
    v = jnp.mean(cen * cen, axis=1, keepdims=True)
    h1 = cen * lax.rsqrt(v + 1e-5) * lng_ref[0] + lnb_ref[0]
    msg = mm(h1, w2_ref) + b2_ref[0]
    cf = _leaky(mm(msg, wc1_ref) + bc1_ref[0])
    cf = jnp.sum(cf * wc2_ref[0], axis=1, keepdims=True) + bc2_ref[0, 0, 0]
    pm_ref[...] = msg
    lane16 = lax.broadcasted_iota(jnp.int32, (1, D_EDGE), 1)
    xd16 = jnp.where(lane16 == 3, 1.0, xs * cf)
    px_ref[...] = jnp.pad(xd16, ((0, 0), (0, D - D_EDGE)))


def _run_edge(out_src, out_dst, ef2, oc_s, oc_d, w1c2, w1d2, b1_2, lng2,
              lnb2, w2_2, b2_2, wc1_2, bc1_2, wc2_2, bc2_2):
    blk = 2048
    nblk = (2 * E) // blk
    half = nblk // 2
    g = lambda i: i // half
    w_dd = pl.BlockSpec((1, D, D), lambda i: (g(i), 0, 0))
    w_de = pl.BlockSpec((1, D, D_EDGE), lambda i: (g(i), 0, 0))
    w_ds = pl.BlockSpec((1, D, NSIG), lambda i: (g(i), 0, 0))
    w_b = pl.BlockSpec((1, 1, D), lambda i: (g(i), 0, 0))
    w_s = pl.BlockSpec((1, 1, 1), lambda i: (g(i), 0, 0))
    espec = pl.BlockSpec((blk, D), lambda i: (i, 0))
    cspec = pl.BlockSpec((blk, D_EDGE), lambda i: (i, 0))
    return pl.pallas_call(
        _edge_body,
        grid=(nblk,),
        in_specs=[
            espec, espec, cspec, cspec, cspec,
            w_de, w_ds, w_b, w_b, w_b, w_dd, w_b, w_dd, w_b, w_b, w_s,
        ],
        out_specs=[espec, espec],
        out_shape=[jax.ShapeDtypeStruct((2 * E, D), jnp.float32),
                   jax.ShapeDtypeStruct((2 * E, D), jnp.float32)],
    )(out_src, out_dst, ef2, oc_s, oc_d, w1c2, w1d2, b1_2, lng2, lnb2,
      w2_2, b2_2, wc1_2, bc1_2, wc2_2, bc2_2)


def _scatter_body(pm_hbm, px_hbm, dm_hbm, zeros_hbm, out_hbm,
                  idm_v, pay_v, shared, sem):
    c = lax.axis_index("c")
    s = lax.axis_index("s")
    wid = c * NS + s
    ebase = wid * EPT
    rows = N // NS
    nj = RCH // GCH

    pltpu.sync_copy(dm_hbm.at[pl.ds(wid * (EPT // GCH), EPT // GCH)], idm_v)

    def half_pass(pay_hbm, out_base):
        pltpu.sync_copy(zeros_hbm.at[pl.ds(s * rows, rows)],
                        shared.at[pl.ds(s * rows, rows)])
        plsc.subcore_barrier()

        def chunk(ch, _):
            rbase = ch * RCH
            pltpu.sync_copy(pay_hbm.at[pl.ds(ebase + rbase, RCH)], pay_v)
            for j in range(nj):
                pltpu.sync_copy(pay_v.at[pl.ds(j * GCH, GCH)],
                                shared.at[idm_v.at[ch * nj + j]], add=True)
            return ()

        lax.fori_loop(0, EPT // RCH, chunk, (), unroll=False)
        plsc.subcore_barrier()
        pltpu.sync_copy(shared.at[pl.ds(s * rows, rows)],
                        out_hbm.at[pl.ds(out_base + s * rows, rows)])
        plsc.subcore_barrier()

    half_pass(pm_hbm, c * 2 * N)
    half_pass(px_hbm, c * 2 * N + N)


def _run_scatter(paym, payx, dstm, zeros_nf):
    mesh = plsc.VectorSubcoreMesh(core_axis_name="c", subcore_axis_name="s")
    k = pl.kernel(
        _scatter_body,
        mesh=mesh,
        out_type=jax.ShapeDtypeStruct((4 * N, D), jnp.float32),
        scratch_types=[
            pltpu.VMEM((EPT // GCH, GCH), jnp.int32),
            pltpu.VMEM((RCH, D), jnp.float32),
            pltpu.VMEM_SHARED((N, D), jnp.float32),
            pltpu.SemaphoreType.DMA,
        ],
        compiler_params=pltpu.CompilerParams(use_tc_tiling_on_sc=True),
    )
    return k(paym, payx, dstm, zeros_nf)


def _att_body(q_ref, kk_ref, vv_ref, o_ref):
    q = q_ref[...]
    k = kk_ref[0]
    v = vv_ref[0]
    l = lax.dot_general(q, k, (((1,), (1,)), ((), ())),
                        preferred_element_type=jnp.float32)
    m = jnp.max(l, axis=1, keepdims=True)
    p = jnp.exp(l - m)
    ssum = jnp.sum(p, axis=1, keepdims=True)
    o = lax.dot_general(p, v, (((1,), (0,)), ((), ())),
                        preferred_element_type=jnp.float32)
    o_ref[...] = o / ssum


def _run_att(q2, kk3, vv3):
    blk = 512
    nblk = (2 * N) // blk
    half = nblk // 2
    return pl.pallas_call(
        _att_body,
        grid=(nblk,),
        in_specs=[
            pl.BlockSpec((blk, D), lambda i: (i, 0)),
            pl.BlockSpec((1, N, D), lambda i: (i // half, 0, 0)),
            pl.BlockSpec((1, N, D), lambda i: (i // half, 0, 0)),
        ],
        out_specs=pl.BlockSpec((blk, D), lambda i: (i, 0)),
        out_shape=jax.ShapeDtypeStruct((2 * N, D), jnp.float32),
    )(q2, kk3, vv3)


def _final_body(sums_ref, xd_ref, h_ref, att_ref, of_ref, cmix_ref,
                wn1_ref, bn1_ref, lng_ref, lnb_ref, wn2_ref, bn2_ref,
                node_ref, xe_ref):
    lane16 = lax.broadcasted_iota(jnp.int32, (1, PW - D), 1)
    coord_mask = jnp.where(lane16 < 3, 1.0, 0.0).astype(jnp.float32)
    mm = lambda x, w: lax.dot_general(x, w, (((1,), (1,)), ((), ())),
                                      preferred_element_type=jnp.float32)
    h = h_ref[...]
    tail = xd_ref[:, 0:(PW - D)]
    deg = jnp.maximum(xd_ref[:, 3:4], 1.0)
    inv_deg = 1.0 / deg
    aggr = sums_ref[...] * inv_deg
    wn1 = wn1_ref[0]
    h1 = (mm(h, wn1[:, 0:D]) + mm(aggr, wn1[:, D:2 * D])
          + mm(att_ref[...], wn1[:, 2 * D:3 * D])
          + mm(of_ref[...], wn1[:, 3 * D:4 * D]) + bn1_ref[0])
    h1 = _leaky(h1)
    m = jnp.mean(h1, axis=1, keepdims=True)
    cen = h1 - m
    v = jnp.mean(cen * cen, axis=1, keepdims=True)
    h1 = cen * lax.rsqrt(v + 1e-5) * lng_ref[0] + lnb_ref[0]
    out = mm(h1, wn2_ref[0]) + bn2_ref[0]
    node_ref[...] = SKIP_H * out + (1.0 - SKIP_H) * h
    xe_ref[...] = cmix_ref[...] + tail * inv_deg * coord_mask


def _run_final(sums, h2, att2, of2, cmix2, wn1_2, bn1_2, lng2, lnb2,
               wn2_2, bn2_2):
    blk = 1024
    nblk = (2 * N) // blk
    half = nblk // 2
    g = lambda i: i // half
    return pl.pallas_call(
        _final_body,
        grid=(nblk,),
        in_specs=[
            pl.BlockSpec((blk, D), lambda i: (g(i) * 8 + i % half, 0)),
            pl.BlockSpec((blk, D), lambda i: (g(i) * 8 + 4 + i % half, 0)),
            pl.BlockSpec((blk, D), lambda i: (i, 0)),
            pl.BlockSpec((blk, D), lambda i: (i, 0)),
            pl.BlockSpec((blk, D), lambda i: (i, 0)),
            pl.BlockSpec((blk, PW - D), lambda i: (i, 0)),
            pl.BlockSpec((1, D, 4 * D), lambda i: (g(i), 0, 0)),
            pl.BlockSpec((1, 1, D), lambda i: (g(i), 0, 0)),
            pl.BlockSpec((1, 1, D), lambda i: (g(i), 0, 0)),
            pl.BlockSpec((1, 1, D), lambda i: (g(i), 0, 0)),
            pl.BlockSpec((1, D, D), lambda i: (g(i), 0, 0)),
            pl.BlockSpec((1, 1, D), lambda i: (g(i), 0, 0)),
        ],
        out_specs=[
            pl.BlockSpec((blk, D), lambda i: (i, 0)),
            pl.BlockSpec((blk, PW - D), lambda i: (i, 0)),
        ],
        out_shape=[
            jax.ShapeDtypeStruct((2 * N, D), jnp.float32),
            jax.ShapeDtypeStruct((2 * N, PW - D), jnp.float32),
        ],
    )(sums, sums, h2, att2, of2, cmix2, wn1_2, bn1_2, lng2, lnb2, wn2_2,
      bn2_2)


def kernel(coords_lig, h_feats_lig, original_ligand_node_features,
           orig_coords_lig, coords_rec, h_feats_rec,
           original_receptor_node_features, orig_coords_rec, edge_feat_lig,
           edge_feat_rec, mask, edge_index_lig, edge_index_rec, params):
    p = params
    f32 = jnp.float32

    h2 = jnp.concatenate([h_feats_lig, h_feats_rec], axis=0)
    coords16 = jnp.concatenate([
        jnp.pad(coords_lig, ((0, 0), (0, D_EDGE - 3))),
        jnp.pad(coords_rec, ((0, 0), (0, D_EDGE - 3))),
    ], axis=0)
    of2 = jnp.concatenate([original_ligand_node_features,
                           original_receptor_node_features], axis=0)
    cmix2 = jnp.concatenate([
        jnp.pad(X_CONN * orig_coords_lig + (1.0 - X_CONN) * coords_lig,
                ((0, 0), (0, PW - D - 3))),
        jnp.pad(X_CONN * orig_coords_rec + (1.0 - X_CONN) * coords_rec,
                ((0, 0), (0, PW - D - 3))),
    ], axis=0)
    ef2 = jnp.concatenate([edge_feat_lig, edge_feat_rec], axis=0)
    src2 = jnp.concatenate([edge_index_lig[0], edge_index_rec[0] + N])
    dst_g = jnp.concatenate([edge_index_lig[1], edge_index_rec[1] + N])
    dstm = jnp.concatenate([edge_index_lig[1],
                            edge_index_rec[1]]).reshape(-1, GCH)

    st = lambda a, b: jnp.stack([a, b])
    stb = lambda a, b: jnp.stack([a, b])[:, None, :]
    le, re = p['lig_edge'], p['rec_edge']
    w1a2 = st(le['W1'][:, 0:D], re['W1'][:, 0:D])
    w1b2 = st(le['W1'][:, D:2 * D], re['W1'][:, D:2 * D])
    w1c2 = st(le['W1'][:, 2 * D:2 * D + D_EDGE], re['W1'][:, 2 * D:2 * D + D_EDGE])
    w1d2 = st(le['W1'][:, 2 * D + D_EDGE:], re['W1'][:, 2 * D + D_EDGE:])
    b1_2 = stb(le['b1'], re['b1'])
    lng2 = stb(le['ln_g'], re['ln_g'])
    lnb2 = stb(le['ln_b'], re['ln_b'])
    w2_2 = st(le['W2'], re['W2'])
    b2_2 = stb(le['b2'], re['b2'])
    lc, rc = p['coords_lig'], p['coords_rec']
    wc1_2 = st(lc['W1'], rc['W1'])
    bc1_2 = stb(lc['b1'], rc['b1'])
    wc2_2 = st(lc['W2'], rc['W2'])
    bc2_2 = st(lc['b2'], rc['b2'])[:, :, None]
    wq2 = st(p['att_Q_lig'], p['att_Q'])
    wk2 = st(p['att_K_lig'], p['att_K'])
    wv2 = st(p['att_V_lig'], p['att_V'])
    nl, nr = p['node_lig'], p['node_rec']
    wn1_2 = st(nl['W1'], nr['W1'])
    bn1_2 = stb(nl['b1'], nr['b1'])
    lngn2 = stb(nl['ln_g'], nr['ln_g'])
    lnbn2 = stb(nl['ln_b'], nr['ln_b'])
    wn2_2 = st(nl['W2'], nr['W2'])
    bn2_2 = stb(nl['b2'], nr['b2'])

    tables_s, tables_d, q2, kk2, vv2 = _run_proj(
        h2, w1a2, w1b2, wq2, wk2, wv2)

    out_src, out_dst = _run_gather(tables_s, tables_d, src2, dst_g)
    oc_s, oc_d = _run_cgather(coords16, src2, dst_g)

    att2 = _run_att(q2, kk2.reshape(2, N, D), vv2.reshape(2, N, D))

    paym, payx = _run_edge(out_src, out_dst, ef2, oc_s, oc_d, w1c2, w1d2,
                           b1_2, lng2, lnb2, w2_2, b2_2, wc1_2, bc1_2,
                           wc2_2, bc2_2)

    zeros_nf = jnp.zeros((N, D), f32)
    sums = _run_scatter(paym, payx, dstm, zeros_nf)

    node2, xe2 = _run_final(sums, h2, att2, of2, cmix2, wn1_2, bn1_2,
                            lngn2, lnbn2, wn2_2, bn2_2)

    return (xe2[:N, 0:3], node2[:N], xe2[N:, 0:3], node2[N:])


# speedup vs baseline: 1.0021x; 1.0021x over previous
import jax
import jax.numpy as jnp
from jax import lax
from jax.experimental import pallas as pl
from jax.experimental.pallas import tpu as pltpu
from jax.experimental.pallas import tpu_sc as plsc

N = 4096
E = 65536
D = 128
D_EDGE = 16
NSIG = 15
_SIGMAS = [1.5 ** x for x in range(15)]
SLOPE = 0.01
SKIP_H = 0.5
X_CONN = 0.25
PW = 144

NC = 2
NS = 16
NW = NC * NS
EPT = (2 * E) // NW
GCH = 128
RCH = 512


def _leaky(x):
    return jnp.where(x >= 0, x, SLOPE * x)


def _proj_body(h_ref, w1a_ref, w1b_ref, wq_ref, wk_ref, wv_ref,
               ts_ref, td_ref, q_ref, kk_ref, vv_ref):
    h = h_ref[...]
    mm = lambda x, wr: lax.dot_general(x, wr[0], (((1,), (1,)), ((), ())),
                                       preferred_element_type=jnp.float32)
    ts_ref[...] = mm(h, w1a_ref)
    td_ref[...] = mm(h, w1b_ref)
    q_ref[...] = _leaky(mm(h, wq_ref))
    kk_ref[...] = _leaky(mm(h, wk_ref))
    vv_ref[...] = mm(h, wv_ref)


def _run_proj(h2, w1a2, w1b2, wq2, wk2, wv2):
    blk = 1024
    nblk = (2 * N) // blk
    half = nblk // 2
    wspec = pl.BlockSpec((1, D, D), lambda i: (i // half, 0, 0))
    nspec = pl.BlockSpec((blk, D), lambda i: (i, 0))
    return pl.pallas_call(
        _proj_body,
        grid=(nblk,),
        in_specs=[nspec, wspec, wspec, wspec, wspec, wspec],
        out_specs=[
            nspec, nspec, nspec,
            pl.BlockSpec((blk, D), lambda i: ((1 - i // half) * half + i % half, 0)),
            pl.BlockSpec((blk, D), lambda i: ((1 - i // half) * half + i % half, 0)),
        ],
        out_shape=[jax.ShapeDtypeStruct((2 * N, D), jnp.float32)] * 5,
    )(h2, w1a2, w1b2, wq2, wk2, wv2)


def _gather_body(ts_hbm, td_hbm, src_hbm, dst_hbm, os_hbm, od_hbm,
                 idxs_v, idxd_v, rows_v, sem):
    c = lax.axis_index("c")
    s = lax.axis_index("s")
    wid = c * NS + s
    ebase = wid * EPT

    pltpu.sync_copy(src_hbm.at[pl.ds(ebase, EPT)], idxs_v)
    pltpu.sync_copy(dst_hbm.at[pl.ds(ebase, EPT)], idxd_v)

    def one_table(tab_hbm, idx_v, out_hbm):
        def chunk(ch, _):
            rbase = ch * RCH
            cps = []
            for j in range(RCH // GCH):
                cps.append(pltpu.async_copy(
                    tab_hbm.at[idx_v.at[pl.ds(rbase + j * GCH, GCH)]],
                    rows_v.at[pl.ds(j * GCH, GCH)], sem))
            for cp in cps:
                cp.wait()
            pltpu.sync_copy(rows_v, out_hbm.at[pl.ds(ebase + rbase, RCH)])
            return ()

        lax.fori_loop(0, EPT // RCH, chunk, (), unroll=False)

    one_table(ts_hbm, idxs_v, os_hbm)
    one_table(td_hbm, idxd_v, od_hbm)


def _run_gather(tables_s, tables_d, src2, dst_g):
    mesh = plsc.VectorSubcoreMesh(core_axis_name="c", subcore_axis_name="s")
    k = pl.kernel(
        _gather_body,
        mesh=mesh,
        out_type=[
            jax.ShapeDtypeStruct((2 * E, D), jnp.float32),
            jax.ShapeDtypeStruct((2 * E, D), jnp.float32),
        ],
        scratch_types=[
            pltpu.VMEM((EPT,), jnp.int32),
            pltpu.VMEM((EPT,), jnp.int32),
            pltpu.VMEM((RCH, D), jnp.float32),
            pltpu.SemaphoreType.DMA,
        ],
        compiler_params=pltpu.CompilerParams(use_tc_tiling_on_sc=True),
    )
    return k(tables_s, tables_d, src2, dst_g)


def _cgather_body(ct_hbm, src_hbm, dst_hbm, ocs_hbm, ocd_hbm,
                  idx_v, rows_v, sem):
    c = lax.axis_index("c")
    s = lax.axis_index("s")
    wid = c * NS + s
    ebase = wid * EPT

    def one_table(eidx_hbm, out_hbm):
        pltpu.sync_copy(eidx_hbm.at[pl.ds(ebase, EPT)], idx_v)

        def chunk(ch, _):
            rbase = ch * RCH
            cps = []
            for j in range(RCH // GCH):
                cps.append(pltpu.async_copy(
                    ct_hbm.at[idx_v.at[pl.ds(rbase + j * GCH, GCH)]],
                    rows_v.at[pl.ds(j * GCH, GCH)], sem))
            for cp in cps:
                cp.wait()
            pltpu.sync_copy(rows_v, out_hbm.at[pl.ds(ebase + rbase, RCH)])
            return ()

        lax.fori_loop(0, EPT // RCH, chunk, (), unroll=False)

    one_table(src_hbm, ocs_hbm)
    one_table(dst_hbm, ocd_hbm)


def _run_cgather(coords16, src2, dst_g):
    mesh = plsc.VectorSubcoreMesh(core_axis_name="c", subcore_axis_name="s")
    k = pl.kernel(
        _cgather_body,
        mesh=mesh,
        out_type=[
            jax.ShapeDtypeStruct((2 * E, D_EDGE), jnp.float32),
            jax.ShapeDtypeStruct((2 * E, D_EDGE), jnp.float32),
        ],
        scratch_types=[
            pltpu.VMEM((EPT,), jnp.int32),
            pltpu.VMEM((RCH, D_EDGE), jnp.float32),
            pltpu.SemaphoreType.DMA,
        ],
        compiler_params=pltpu.CompilerParams(use_tc_tiling_on_sc=False),
    )
    return k(coords16, src2, dst_g)


def _edge_body(os_ref, od_ref, ef_ref, ocs_ref, ocd_ref, w1c_ref, w1d_ref,
               b1_ref, lng_ref, lnb_ref, w2_ref, b2_ref, wc1_ref, bc1_ref,
               wc2_ref, bc2_ref, pm_ref, px_ref):
    ksig = lax.broadcasted_iota(jnp.int32, (1, NSIG), 1).astype(jnp.float32)
    inv_sig = jnp.exp(-ksig * jnp.log(jnp.float32(1.5)))

    mm = lambda x, wr: lax.dot_general(x, wr[0], (((1,), (1,)), ((), ())),
                                       preferred_element_type=jnp.float32)
    xs = ocs_ref[...] - ocd_ref[...]
    sq = jnp.sum(xs * xs, axis=1, keepdims=True)
    rbf = jnp.exp(-sq * inv_sig)
    h1 = (os_ref[...] + od_ref[...] + mm(ef_ref[...], w1c_ref)
          + mm(rbf, w1d_ref) + b1_ref[0])
    h1 = _leaky(h1)
    m = jnp.mean(h1, axis=1, keepdims=True)
    cen = h1 - m
    v = jnp.mean(cen * cen, axis=1, keepdims=True)
    h1 = cen * lax.rsqrt(v + 1e-5) * lng_ref[0] + lnb_ref[0]
    msg = mm(h1, w2_ref) + b2_ref[0]
    cf = _leaky(mm(msg, wc1_ref) + bc1_ref[0])
    cf = jnp.sum(cf * wc2_ref[0], axis=1, keepdims=True) + bc2_ref[0, 0, 0]
    pm_ref[...] = msg
    lane16 = lax.broadcasted_iota(jnp.int32, (1, D_EDGE), 1)
    xd16 = jnp.where(lane16 == 3, 1.0, xs * cf)
    px_ref[...] = jnp.pad(xd16, ((0, 0), (0, D - D_EDGE)))


def _run_edge(out_src, out_dst, ef2, oc_s, oc_d, w1c2, w1d2, b1_2, lng2,
              lnb2, w2_2, b2_2, wc1_2, bc1_2, wc2_2, bc2_2):
    blk = 2048
    nblk = (2 * E) // blk
    half = nblk // 2
    g = lambda i: i // half
    w_dd = pl.BlockSpec((1, D, D), lambda i: (g(i), 0, 0))
    w_de = pl.BlockSpec((1, D, D_EDGE), lambda i: (g(i), 0, 0))
    w_ds = pl.BlockSpec((1, D, NSIG), lambda i: (g(i), 0, 0))
    w_b = pl.BlockSpec((1, 1, D), lambda i: (g(i), 0, 0))
    w_s = pl.BlockSpec((1, 1, 1), lambda i: (g(i), 0, 0))
    espec = pl.BlockSpec((blk, D), lambda i: (i, 0))
    cspec = pl.BlockSpec((blk, D_EDGE), lambda i: (i, 0))
    return pl.pallas_call(
        _edge_body,
        grid=(nblk,),
        in_specs=[
            espec, espec, cspec, cspec, cspec,
            w_de, w_ds, w_b, w_b, w_b, w_dd, w_b, w_dd, w_b, w_b, w_s,
        ],
        out_specs=[espec, espec],
        out_shape=[jax.ShapeDtypeStruct((2 * E, D), jnp.float32),
                   jax.ShapeDtypeStruct((2 * E, D), jnp.float32)],
    )(out_src, out_dst, ef2, oc_s, oc_d, w1c2, w1d2, b1_2, lng2, lnb2,
      w2_2, b2_2, wc1_2, bc1_2, wc2_2, bc2_2)


def _scatter_body(pm_hbm, px_hbm, dm_hbm, zeros_hbm, out_hbm,
                  idm_v, pay_v, shared, sem):
    c = lax.axis_index("c")
    s = lax.axis_index("s")
    wid = c * NS + s
    ebase = wid * EPT
    rows = N // NS
    nj = RCH // GCH

    pltpu.sync_copy(dm_hbm.at[pl.ds(wid * (EPT // GCH), EPT // GCH)], idm_v)

    def half_pass(pay_hbm, out_base):
        pltpu.sync_copy(zeros_hbm.at[pl.ds(s * rows, rows)],
                        shared.at[pl.ds(s * rows, rows)])
        plsc.subcore_barrier()

        def chunk(ch, _):
            rbase = ch * RCH
            pltpu.sync_copy(pay_hbm.at[pl.ds(ebase + rbase, RCH)], pay_v)
            for j in range(nj):
                pltpu.sync_copy(pay_v.at[pl.ds(j * GCH, GCH)],
                                shared.at[idm_v.at[ch * nj + j]], add=True)
            return ()

        lax.fori_loop(0, EPT // RCH, chunk, (), unroll=False)
        plsc.subcore_barrier()
        pltpu.sync_copy(shared.at[pl.ds(s * rows, rows)],
                        out_hbm.at[pl.ds(out_base + s * rows, rows)])
        plsc.subcore_barrier()

    half_pass(pm_hbm, c * 2 * N)
    half_pass(px_hbm, c * 2 * N + N)


def _run_scatter(paym, payx, dstm, zeros_nf):
    mesh = plsc.VectorSubcoreMesh(core_axis_name="c", subcore_axis_name="s")
    k = pl.kernel(
        _scatter_body,
        mesh=mesh,
        out_type=jax.ShapeDtypeStruct((4 * N, D), jnp.float32),
        scratch_types=[
            pltpu.VMEM((EPT // GCH, GCH), jnp.int32),
            pltpu.VMEM((RCH, D), jnp.float32),
            pltpu.VMEM_SHARED((N, D), jnp.float32),
            pltpu.SemaphoreType.DMA,
        ],
        compiler_params=pltpu.CompilerParams(use_tc_tiling_on_sc=True),
    )
    return k(paym, payx, dstm, zeros_nf)


def _att_body(q_ref, kk_ref, vv_ref, o_ref):
    q = q_ref[...]
    k = kk_ref[0]
    v = vv_ref[0]
    l = lax.dot_general(q, k, (((1,), (1,)), ((), ())),
                        preferred_element_type=jnp.float32)
    m = jnp.max(l, axis=1, keepdims=True)
    p = jnp.exp(l - m)
    ssum = jnp.sum(p, axis=1, keepdims=True)
    o = lax.dot_general(p, v, (((1,), (0,)), ((), ())),
                        preferred_element_type=jnp.float32)
    o_ref[...] = o / ssum


def _run_att(q2, kk3, vv3):
    blk = 512
    nblk = (2 * N) // blk
    half = nblk // 2
    return pl.pallas_call(
        _att_body,
        grid=(nblk,),
        in_specs=[
            pl.BlockSpec((blk, D), lambda i: (i, 0)),
            pl.BlockSpec((1, N, D), lambda i: (i // half, 0, 0)),
            pl.BlockSpec((1, N, D), lambda i: (i // half, 0, 0)),
        ],
        out_specs=pl.BlockSpec((blk, D), lambda i: (i, 0)),
        out_shape=jax.ShapeDtypeStruct((2 * N, D), jnp.float32),
    )(q2, kk3, vv3)


def _final_body(sums_ref, xd_ref, h_ref, att_ref, of_ref, cmix_ref,
                wn1_ref, bn1_ref, lng_ref, lnb_ref, wn2_ref, bn2_ref,
                node_ref, xe_ref):
    lane16 = lax.broadcasted_iota(jnp.int32, (1, PW - D), 1)
    coord_mask = jnp.where(lane16 < 3, 1.0, 0.0).astype(jnp.float32)
    mm = lambda x, w: lax.dot_general(x, w, (((1,), (1,)), ((), ())),
                                      preferred_element_type=jnp.float32)
    h = h_ref[...]
    tail = xd_ref[:, 0:(PW - D)]
    deg = jnp.maximum(xd_ref[:, 3:4], 1.0)
    inv_deg = 1.0 / deg
    aggr = sums_ref[...] * inv_deg
    wn1 = wn1_ref[0]
    h1 = (mm(h, wn1[:, 0:D]) + mm(aggr, wn1[:, D:2 * D])
          + mm(att_ref[...], wn1[:, 2 * D:3 * D])
          + mm(of_ref[...], wn1[:, 3 * D:4 * D]) + bn1_ref[0])
    h1 = _leaky(h1)
    m = jnp.mean(h1, axis=1, keepdims=True)
    cen = h1 - m
    v = jnp.mean(cen * cen, axis=1, keepdims=True)
    h1 = cen * lax.rsqrt(v + 1e-5) * lng_ref[0] + lnb_ref[0]
    out = mm(h1, wn2_ref[0]) + bn2_ref[0]
    node_ref[...] = SKIP_H * out + (1.0 - SKIP_H) * h
    xe_ref[...] = cmix_ref[...] + tail * inv_deg * coord_mask


def _run_final(sums, h2, att2, of2, cmix2, wn1_2, bn1_2, lng2, lnb2,
               wn2_2, bn2_2):
    blk = 1024
    nblk = (2 * N) // blk
    half = nblk // 2
    g = lambda i: i // half
    return pl.pallas_call(
        _final_body,
        grid=(nblk,),
        in_specs=[
            pl.BlockSpec((blk, D), lambda i: (g(i) * 8 + i % half, 0)),
            pl.BlockSpec((blk, D), lambda i: (g(i) * 8 + 4 + i % half, 0)),
            pl.BlockSpec((blk, D), lambda i: (i, 0)),
            pl.BlockSpec((blk, D), lambda i: (i, 0)),
            pl.BlockSpec((blk, D), lambda i: (i, 0)),
            pl.BlockSpec((blk, PW - D), lambda i: (i, 0)),
            pl.BlockSpec((1, D, 4 * D), lambda i: (g(i), 0, 0)),
            pl.BlockSpec((1, 1, D), lambda i: (g(i), 0, 0)),
            pl.BlockSpec((1, 1, D), lambda i: (g(i), 0, 0)),
            pl.BlockSpec((1, 1, D), lambda i: (g(i), 0, 0)),
            pl.BlockSpec((1, D, D), lambda i: (g(i), 0, 0)),
            pl.BlockSpec((1, 1, D), lambda i: (g(i), 0, 0)),
        ],
        out_specs=[
            pl.BlockSpec((blk, D), lambda i: (i, 0)),
            pl.BlockSpec((blk, PW - D), lambda i: (i, 0)),
        ],
        out_shape=[
            jax.ShapeDtypeStruct((2 * N, D), jnp.float32),
            jax.ShapeDtypeStruct((2 * N, PW - D), jnp.float32),
        ],
    )(sums, sums, h2, att2, of2, cmix2, wn1_2, bn1_2, lng2, lnb2, wn2_2,
      bn2_2)


def kernel(coords_lig, h_feats_lig, original_ligand_node_features,
           orig_coords_lig, coords_rec, h_feats_rec,
           original_receptor_node_features, orig_coords_rec, edge_feat_lig,
           edge_feat_rec, mask, edge_index_lig, edge_index_rec, params):
    p = params
    f32 = jnp.float32

    h2 = jnp.concatenate([h_feats_lig, h_feats_rec], axis=0)
    coords16 = jnp.concatenate([
        jnp.pad(coords_lig, ((0, 0), (0, D_EDGE - 3))),
        jnp.pad(coords_rec, ((0, 0), (0, D_EDGE - 3))),
    ], axis=0)
    of2 = jnp.concatenate([original_ligand_node_features,
                           original_receptor_node_features], axis=0)
    cmix2 = jnp.concatenate([
        jnp.pad(X_CONN * orig_coords_lig + (1.0 - X_CONN) * coords_lig,
                ((0, 0), (0, PW - D - 3))),
        jnp.pad(X_CONN * orig_coords_rec + (1.0 - X_CONN) * coords_rec,
                ((0, 0), (0, PW - D - 3))),
    ], axis=0)
    ef2 = jnp.concatenate([edge_feat_lig, edge_feat_rec], axis=0)
    src2 = jnp.concatenate([edge_index_lig[0], edge_index_rec[0] + N])
    dst_g = jnp.concatenate([edge_index_lig[1], edge_index_rec[1] + N])
    dstm = jnp.concatenate([edge_index_lig[1],
                            edge_index_rec[1]]).reshape(-1, GCH)

    st = lambda a, b: jnp.stack([a, b])
    stb = lambda a, b: jnp.stack([a, b])[:, None, :]
    le, re = p['lig_edge'], p['rec_edge']
    w1a2 = st(le['W1'][:, 0:D], re['W1'][:, 0:D])
    w1b2 = st(le['W1'][:, D:2 * D], re['W1'][:, D:2 * D])
    w1c2 = st(le['W1'][:, 2 * D:2 * D + D_EDGE], re['W1'][:, 2 * D:2 * D + D_EDGE])
    w1d2 = st(le['W1'][:, 2 * D + D_EDGE:], re['W1'][:, 2 * D + D_EDGE:])
    b1_2 = stb(le['b1'], re['b1'])
    lng2 = stb(le['ln_g'], re['ln_g'])
    lnb2 = stb(le['ln_b'], re['ln_b'])
    w2_2 = st(le['W2'], re['W2'])
    b2_2 = stb(le['b2'], re['b2'])
    lc, rc = p['coords_lig'], p['coords_rec']
    wc1_2 = st(lc['W1'], rc['W1'])
    bc1_2 = stb(lc['b1'], rc['b1'])
    wc2_2 = st(lc['W2'], rc['W2'])
    bc2_2 = st(lc['b2'], rc['b2'])[:, :, None]
    wq2 = st(p['att_Q_lig'], p['att_Q'])
    wk2 = st(p['att_K_lig'], p['att_K'])
    wv2 = st(p['att_V_lig'], p['att_V'])
    nl, nr = p['node_lig'], p['node_rec']
    wn1_2 = st(nl['W1'], nr['W1'])
    bn1_2 = stb(nl['b1'], nr['b1'])
    lngn2 = stb(nl['ln_g'], nr['ln_g'])
    lnbn2 = stb(nl['ln_b'], nr['ln_b'])
    wn2_2 = st(nl['W2'], nr['W2'])
    bn2_2 = stb(nl['b2'], nr['b2'])

    tables_s, tables_d, q2, kk2, vv2 = _run_proj(
        h2, w1a2, w1b2, wq2, wk2, wv2)

    out_src, out_dst = _run_gather(tables_s, tables_d, src2, dst_g)
    oc_s, oc_d = _run_cgather(coords16, src2, dst_g)

    att2 = _run_att(q2, kk2.reshape(2, N, D), vv2.reshape(2, N, D))

    paym, payx = _run_edge(out_src, out_dst, ef2, oc_s, oc_d, w1c2, w1d2,
                           b1_2, lng2, lnb2, w2_2, b2_2, wc1_2, bc1_2,
                           wc2_2, bc2_2)

    zeros_nf = jnp.zeros((N, D), f32)
    sums = _run_scatter(paym, payx, dstm, zeros_nf)

    node2, xe2 = _run_final(sums, h2, att2, of2, cmix2, wn1_2, bn1_2,
                            lngn2, lnbn2, wn2_2, bn2_2)

    return (xe2[:N, 0:3], node2[:N], xe2[N:, 0:3], node2[N:])


# speedup vs baseline: 1.0894x; 1.0871x over previous
import jax
import jax.numpy as jnp
from jax import lax
from jax.experimental import pallas as pl
from jax.experimental.pallas import tpu as pltpu
from jax.experimental.pallas import tpu_sc as plsc

N = 4096
E = 65536
D = 128
D_EDGE = 16
NSIG = 15
_SIGMAS = [1.5 ** x for x in range(15)]
SLOPE = 0.01
SKIP_H = 0.5
X_CONN = 0.25
PW = 144

NC = 2
NS = 16
NW = NC * NS
EPT = (2 * E) // NW
GCH = 128
RCH = 512


def _leaky(x):
    return jnp.where(x >= 0, x, SLOPE * x)


def _proj_body(h_ref, w1a_ref, w1b_ref, wq_ref, wk_ref, wv_ref,
               ts_ref, td_ref, q_ref, kk_ref, vv_ref):
    h = h_ref[...]
    mm = lambda x, wr: lax.dot_general(x, wr[0], (((1,), (1,)), ((), ())),
                                       preferred_element_type=jnp.float32)
    ts_ref[...] = mm(h, w1a_ref)
    td_ref[...] = mm(h, w1b_ref)
    q_ref[...] = _leaky(mm(h, wq_ref))
    kk_ref[...] = _leaky(mm(h, wk_ref))
    vv_ref[...] = mm(h, wv_ref)


def _run_proj(h2, w1a2, w1b2, wq2, wk2, wv2):
    blk = 1024
    nblk = (2 * N) // blk
    half = nblk // 2
    wspec = pl.BlockSpec((1, D, D), lambda i: (i // half, 0, 0))
    nspec = pl.BlockSpec((blk, D), lambda i: (i, 0))
    return pl.pallas_call(
        _proj_body,
        grid=(nblk,),
        in_specs=[nspec, wspec, wspec, wspec, wspec, wspec],
        out_specs=[
            nspec, nspec, nspec,
            pl.BlockSpec((blk, D), lambda i: ((1 - i // half) * half + i % half, 0)),
            pl.BlockSpec((blk, D), lambda i: ((1 - i // half) * half + i % half, 0)),
        ],
        out_shape=[jax.ShapeDtypeStruct((2 * N, D), jnp.float32)] * 5,
    )(h2, w1a2, w1b2, wq2, wk2, wv2)


def _gather_body(ts_hbm, td_hbm, src_hbm, dst_hbm, os_hbm, od_hbm,
                 idxs_v, idxd_v, rows_v, sem):
    c = lax.axis_index("c")
    s = lax.axis_index("s")
    wid = c * NS + s
    ebase = wid * EPT

    pltpu.sync_copy(src_hbm.at[pl.ds(ebase, EPT)], idxs_v)
    pltpu.sync_copy(dst_hbm.at[pl.ds(ebase, EPT)], idxd_v)

    def one_table(tab_hbm, idx_v, out_hbm):
        def chunk(ch, _):
            rbase = ch * RCH
            cps = []
            for j in range(RCH // GCH):
                cps.append(pltpu.async_copy(
                    tab_hbm.at[idx_v.at[pl.ds(rbase + j * GCH, GCH)]],
                    rows_v.at[pl.ds(j * GCH, GCH)], sem))
            for cp in cps:
                cp.wait()
            pltpu.sync_copy(rows_v, out_hbm.at[pl.ds(ebase + rbase, RCH)])
            return ()

        lax.fori_loop(0, EPT // RCH, chunk, (), unroll=False)

    one_table(ts_hbm, idxs_v, os_hbm)
    one_table(td_hbm, idxd_v, od_hbm)


def _run_gather(tables_s, tables_d, src2, dst_g):
    mesh = plsc.VectorSubcoreMesh(core_axis_name="c", subcore_axis_name="s")
    k = pl.kernel(
        _gather_body,
        mesh=mesh,
        out_type=[
            jax.ShapeDtypeStruct((2 * E, D), jnp.float32),
            jax.ShapeDtypeStruct((2 * E, D), jnp.float32),
        ],
        scratch_types=[
            pltpu.VMEM((EPT,), jnp.int32),
            pltpu.VMEM((EPT,), jnp.int32),
            pltpu.VMEM((RCH, D), jnp.float32),
            pltpu.SemaphoreType.DMA,
        ],
        compiler_params=pltpu.CompilerParams(use_tc_tiling_on_sc=True),
    )
    return k(tables_s, tables_d, src2, dst_g)


def _cgather_body(ct_hbm, src_hbm, dst_hbm, xs_hbm,
                  idxs_v, idxd_v, rows_v, rows_v2, sem):
    c = lax.axis_index("c")
    s = lax.axis_index("s")
    wid = c * NS + s
    ebase = wid * EPT

    pltpu.sync_copy(src_hbm.at[pl.ds(ebase, EPT)], idxs_v)
    pltpu.sync_copy(dst_hbm.at[pl.ds(ebase, EPT)], idxd_v)

    def chunk(ch, _):
        rbase = ch * RCH
        cps = []
        for j in range(RCH // GCH):
            cps.append(pltpu.async_copy(
                ct_hbm.at[idxs_v.at[pl.ds(rbase + j * GCH, GCH)]],
                rows_v.at[pl.ds(j * GCH, GCH)], sem))
            cps.append(pltpu.async_copy(
                ct_hbm.at[idxd_v.at[pl.ds(rbase + j * GCH, GCH)]],
                rows_v2.at[pl.ds(j * GCH, GCH)], sem))
        for cp in cps:
            cp.wait()

        def sub(r, _):
            rows_v[r, :] = rows_v[r, :] - rows_v2[r, :]
            return ()

        lax.fori_loop(0, RCH, sub, (), unroll=False)
        pltpu.sync_copy(rows_v, xs_hbm.at[pl.ds(ebase + rbase, RCH)])
        return ()

    lax.fori_loop(0, EPT // RCH, chunk, (), unroll=False)


def _run_cgather(coords16, src2, dst_g):
    mesh = plsc.VectorSubcoreMesh(core_axis_name="c", subcore_axis_name="s")
    k = pl.kernel(
        _cgather_body,
        mesh=mesh,
        out_type=jax.ShapeDtypeStruct((2 * E, D_EDGE), jnp.float32),
        scratch_types=[
            pltpu.VMEM((EPT,), jnp.int32),
            pltpu.VMEM((EPT,), jnp.int32),
            pltpu.VMEM((RCH, D_EDGE), jnp.float32),
            pltpu.VMEM((RCH, D_EDGE), jnp.float32),
            pltpu.SemaphoreType.DMA,
        ],
        compiler_params=pltpu.CompilerParams(use_tc_tiling_on_sc=False),
    )
    return k(coords16, src2, dst_g)


def _edge_body(os_ref, od_ref, ef_ref, xs_ref, w1c_ref, w1d_ref,
               b1_ref, lng_ref, lnb_ref, w2_ref, b2_ref, wc1_ref, bc1_ref,
               wc2_ref, bc2_ref, pm_ref, px_ref):
    ksig = lax.broadcasted_iota(jnp.int32, (1, NSIG), 1).astype(jnp.float32)
    inv_sig = jnp.exp(-ksig * jnp.log(jnp.float32(1.5)))

    mm = lambda x, wr: lax.dot_general(x, wr[0], (((1,), (1,)), ((), ())),
                                       preferred_element_type=jnp.float32)
    xs = xs_ref[...]
    sq = jnp.sum(xs * xs, axis=1, keepdims=True)
    rbf = jnp.exp(-sq * inv_sig)
    h1 = (os_ref[...] + od_ref[...] + mm(ef_ref[...], w1c_ref)
          + mm(rbf, w1d_ref) + b1_ref[0])
    h1 = _leaky(h1)
    m = jnp.mean(h1, axis=1, keepdims=True)
    cen = h1 - m
    v = jnp.mean(cen * cen, axis=1, keepdims=True)
    h1 = cen * lax.rsqrt(v + 1e-5) * lng_ref[0] + lnb_ref[0]
    msg = mm(h1, w2_ref) + b2_ref[0]
    cf = _leaky(mm(msg, wc1_ref) + bc1_ref[0])
    cf = jnp.sum(cf * wc2_ref[0], axis=1, keepdims=True) + bc2_ref[0, 0, 0]
    pm_ref[...] = msg
    lane16 = lax.broadcasted_iota(jnp.int32, (1, D_EDGE), 1)
    xd16 = jnp.where(lane16 == 3, 1.0, xs * cf)
    px_ref[...] = jnp.pad(xd16, ((0, 0), (0, D - D_EDGE)))


def _run_edge(out_src, out_dst, ef2, xs2, w1c2, w1d2, b1_2, lng2,
              lnb2, w2_2, b2_2, wc1_2, bc1_2, wc2_2, bc2_2):
    blk = 2048
    nblk = (2 * E) // blk
    half = nblk // 2
    g = lambda i: i // half
    w_dd = pl.BlockSpec((1, D, D), lambda i: (g(i), 0, 0))
    w_de = pl.BlockSpec((1, D, D_EDGE), lambda i: (g(i), 0, 0))
    w_ds = pl.BlockSpec((1, D, NSIG), lambda i: (g(i), 0, 0))
    w_b = pl.BlockSpec((1, 1, D), lambda i: (g(i), 0, 0))
    w_s = pl.BlockSpec((1, 1, 1), lambda i: (g(i), 0, 0))
    espec = pl.BlockSpec((blk, D), lambda i: (i, 0))
    cspec = pl.BlockSpec((blk, D_EDGE), lambda i: (i, 0))
    return pl.pallas_call(
        _edge_body,
        grid=(nblk,),
        in_specs=[
            espec, espec, cspec, cspec,
            w_de, w_ds, w_b, w_b, w_b, w_dd, w_b, w_dd, w_b, w_b, w_s,
        ],
        out_specs=[espec, espec],
        out_shape=[jax.ShapeDtypeStruct((2 * E, D), jnp.float32),
                   jax.ShapeDtypeStruct((2 * E, D), jnp.float32)],
    )(out_src, out_dst, ef2, xs2, w1c2, w1d2, b1_2, lng2, lnb2,
      w2_2, b2_2, wc1_2, bc1_2, wc2_2, bc2_2)


def _scatter_body(pm_hbm, px_hbm, dm_hbm, zeros_hbm, out_hbm,
                  idm_v, pay_v, shared, sem):
    c = lax.axis_index("c")
    s = lax.axis_index("s")
    wid = c * NS + s
    ebase = wid * EPT
    rows = N // NS
    nj = RCH // GCH

    pltpu.sync_copy(dm_hbm.at[pl.ds(wid * (EPT // GCH), EPT // GCH)], idm_v)

    def half_pass(pay_hbm, out_base):
        pltpu.sync_copy(zeros_hbm.at[pl.ds(s * rows, rows)],
                        shared.at[pl.ds(s * rows, rows)])
        plsc.subcore_barrier()

        def chunk(ch, _):
            rbase = ch * RCH
            pltpu.sync_copy(pay_hbm.at[pl.ds(ebase + rbase, RCH)], pay_v)
            for j in range(nj):
                pltpu.sync_copy(pay_v.at[pl.ds(j * GCH, GCH)],
                                shared.at[idm_v.at[ch * nj + j]], add=True)
            return ()

        lax.fori_loop(0, EPT // RCH, chunk, (), unroll=False)
        plsc.subcore_barrier()
        pltpu.sync_copy(shared.at[pl.ds(s * rows, rows)],
                        out_hbm.at[pl.ds(out_base + s * rows, rows)])
        plsc.subcore_barrier()

    half_pass(pm_hbm, c * 2 * N)
    half_pass(px_hbm, c * 2 * N + N)


def _run_scatter(paym, payx, dstm, zeros_nf):
    mesh = plsc.VectorSubcoreMesh(core_axis_name="c", subcore_axis_name="s")
    k = pl.kernel(
        _scatter_body,
        mesh=mesh,
        out_type=jax.ShapeDtypeStruct((4 * N, D), jnp.float32),
        scratch_types=[
            pltpu.VMEM((EPT // GCH, GCH), jnp.int32),
            pltpu.VMEM((RCH, D), jnp.float32),
            pltpu.VMEM_SHARED((N, D), jnp.float32),
            pltpu.SemaphoreType.DMA,
        ],
        compiler_params=pltpu.CompilerParams(use_tc_tiling_on_sc=True),
    )
    return k(paym, payx, dstm, zeros_nf)


def _att_body(q_ref, kk_ref, vv_ref, o_ref):
    q = q_ref[...]
    k = kk_ref[0]
    v = vv_ref[0]
    l = lax.dot_general(q, k, (((1,), (1,)), ((), ())),
                        preferred_element_type=jnp.float32)
    m = jnp.max(l, axis=1, keepdims=True)
    p = jnp.exp(l - m)
    ssum = jnp.sum(p, axis=1, keepdims=True)
    o = lax.dot_general(p, v, (((1,), (0,)), ((), ())),
                        preferred_element_type=jnp.float32)
    o_ref[...] = o / ssum


def _run_att(q2, kk3, vv3):
    blk = 512
    nblk = (2 * N) // blk
    half = nblk // 2
    return pl.pallas_call(
        _att_body,
        grid=(nblk,),
        in_specs=[
            pl.BlockSpec((blk, D), lambda i: (i, 0)),
            pl.BlockSpec((1, N, D), lambda i: (i // half, 0, 0)),
            pl.BlockSpec((1, N, D), lambda i: (i // half, 0, 0)),
        ],
        out_specs=pl.BlockSpec((blk, D), lambda i: (i, 0)),
        out_shape=jax.ShapeDtypeStruct((2 * N, D), jnp.float32),
    )(q2, kk3, vv3)


def _final_body(sums_ref, xd_ref, h_ref, att_ref, of_ref, cmix_ref,
                wn1_ref, bn1_ref, lng_ref, lnb_ref, wn2_ref, bn2_ref,
                node_ref, xe_ref):
    lane16 = lax.broadcasted_iota(jnp.int32, (1, PW - D), 1)
    coord_mask = jnp.where(lane16 < 3, 1.0, 0.0).astype(jnp.float32)
    mm = lambda x, w: lax.dot_general(x, w, (((1,), (1,)), ((), ())),
                                      preferred_element_type=jnp.float32)
    h = h_ref[...]
    tail = xd_ref[:, 0:(PW - D)]
    deg = jnp.maximum(xd_ref[:, 3:4], 1.0)
    inv_deg = 1.0 / deg
    aggr = sums_ref[...] * inv_deg
    wn1 = wn1_ref[0]
    h1 = (mm(h, wn1[:, 0:D]) + mm(aggr, wn1[:, D:2 * D])
          + mm(att_ref[...], wn1[:, 2 * D:3 * D])
          + mm(of_ref[...], wn1[:, 3 * D:4 * D]) + bn1_ref[0])
    h1 = _leaky(h1)
    m = jnp.mean(h1, axis=1, keepdims=True)
    cen = h1 - m
    v = jnp.mean(cen * cen, axis=1, keepdims=True)
    h1 = cen * lax.rsqrt(v + 1e-5) * lng_ref[0] + lnb_ref[0]
    out = mm(h1, wn2_ref[0]) + bn2_ref[0]
    node_ref[...] = SKIP_H * out + (1.0 - SKIP_H) * h
    xe_ref[...] = cmix_ref[...] + tail * inv_deg * coord_mask


def _run_final(sums, h2, att2, of2, cmix2, wn1_2, bn1_2, lng2, lnb2,
               wn2_2, bn2_2):
    blk = 1024
    nblk = (2 * N) // blk
    half = nblk // 2
    g = lambda i: i // half
    return pl.pallas_call(
        _final_body,
        grid=(nblk,),
        in_specs=[
            pl.BlockSpec((blk, D), lambda i: (g(i) * 8 + i % half, 0)),
            pl.BlockSpec((blk, D), lambda i: (g(i) * 8 + 4 + i % half, 0)),
            pl.BlockSpec((blk, D), lambda i: (i, 0)),
            pl.BlockSpec((blk, D), lambda i: (i, 0)),
            pl.BlockSpec((blk, D), lambda i: (i, 0)),
            pl.BlockSpec((blk, PW - D), lambda i: (i, 0)),
            pl.BlockSpec((1, D, 4 * D), lambda i: (g(i), 0, 0)),
            pl.BlockSpec((1, 1, D), lambda i: (g(i), 0, 0)),
            pl.BlockSpec((1, 1, D), lambda i: (g(i), 0, 0)),
            pl.BlockSpec((1, 1, D), lambda i: (g(i), 0, 0)),
            pl.BlockSpec((1, D, D), lambda i: (g(i), 0, 0)),
            pl.BlockSpec((1, 1, D), lambda i: (g(i), 0, 0)),
        ],
        out_specs=[
            pl.BlockSpec((blk, D), lambda i: (i, 0)),
            pl.BlockSpec((blk, PW - D), lambda i: (i, 0)),
        ],
        out_shape=[
            jax.ShapeDtypeStruct((2 * N, D), jnp.float32),
            jax.ShapeDtypeStruct((2 * N, PW - D), jnp.float32),
        ],
    )(sums, sums, h2, att2, of2, cmix2, wn1_2, bn1_2, lng2, lnb2, wn2_2,
      bn2_2)


def kernel(coords_lig, h_feats_lig, original_ligand_node_features,
           orig_coords_lig, coords_rec, h_feats_rec,
           original_receptor_node_features, orig_coords_rec, edge_feat_lig,
           edge_feat_rec, mask, edge_index_lig, edge_index_rec, params):
    p = params
    f32 = jnp.float32

    h2 = jnp.concatenate([h_feats_lig, h_feats_rec], axis=0)
    coords16 = jnp.concatenate([
        jnp.pad(coords_lig, ((0, 0), (0, D_EDGE - 3))),
        jnp.pad(coords_rec, ((0, 0), (0, D_EDGE - 3))),
    ], axis=0)
    of2 = jnp.concatenate([original_ligand_node_features,
                           original_receptor_node_features], axis=0)
    cmix2 = jnp.concatenate([
        jnp.pad(X_CONN * orig_coords_lig + (1.0 - X_CONN) * coords_lig,
                ((0, 0), (0, PW - D - 3))),
        jnp.pad(X_CONN * orig_coords_rec + (1.0 - X_CONN) * coords_rec,
                ((0, 0), (0, PW - D - 3))),
    ], axis=0)
    ef2 = jnp.concatenate([edge_feat_lig, edge_feat_rec], axis=0)
    src2 = jnp.concatenate([edge_index_lig[0], edge_index_rec[0] + N])
    dst_g = jnp.concatenate([edge_index_lig[1], edge_index_rec[1] + N])
    dstm = jnp.concatenate([edge_index_lig[1],
                            edge_index_rec[1]]).reshape(-1, GCH)

    st = lambda a, b: jnp.stack([a, b])
    stb = lambda a, b: jnp.stack([a, b])[:, None, :]
    le, re = p['lig_edge'], p['rec_edge']
    w1a2 = st(le['W1'][:, 0:D], re['W1'][:, 0:D])
    w1b2 = st(le['W1'][:, D:2 * D], re['W1'][:, D:2 * D])
    w1c2 = st(le['W1'][:, 2 * D:2 * D + D_EDGE], re['W1'][:, 2 * D:2 * D + D_EDGE])
    w1d2 = st(le['W1'][:, 2 * D + D_EDGE:], re['W1'][:, 2 * D + D_EDGE:])
    b1_2 = stb(le['b1'], re['b1'])
    lng2 = stb(le['ln_g'], re['ln_g'])
    lnb2 = stb(le['ln_b'], re['ln_b'])
    w2_2 = st(le['W2'], re['W2'])
    b2_2 = stb(le['b2'], re['b2'])
    lc, rc = p['coords_lig'], p['coords_rec']
    wc1_2 = st(lc['W1'], rc['W1'])
    bc1_2 = stb(lc['b1'], rc['b1'])
    wc2_2 = st(lc['W2'], rc['W2'])
    bc2_2 = st(lc['b2'], rc['b2'])[:, :, None]
    wq2 = st(p['att_Q_lig'], p['att_Q'])
    wk2 = st(p['att_K_lig'], p['att_K'])
    wv2 = st(p['att_V_lig'], p['att_V'])
    nl, nr = p['node_lig'], p['node_rec']
    wn1_2 = st(nl['W1'], nr['W1'])
    bn1_2 = stb(nl['b1'], nr['b1'])
    lngn2 = stb(nl['ln_g'], nr['ln_g'])
    lnbn2 = stb(nl['ln_b'], nr['ln_b'])
    wn2_2 = st(nl['W2'], nr['W2'])
    bn2_2 = stb(nl['b2'], nr['b2'])

    tables_s, tables_d, q2, kk2, vv2 = _run_proj(
        h2, w1a2, w1b2, wq2, wk2, wv2)

    out_src, out_dst = _run_gather(tables_s, tables_d, src2, dst_g)
    xs2 = _run_cgather(coords16, src2, dst_g)

    att2 = _run_att(q2, kk2.reshape(2, N, D), vv2.reshape(2, N, D))

    paym, payx = _run_edge(out_src, out_dst, ef2, xs2, w1c2, w1d2,
                           b1_2, lng2, lnb2, w2_2, b2_2, wc1_2, bc1_2,
                           wc2_2, bc2_2)

    zeros_nf = jnp.zeros((N, D), f32)
    sums = _run_scatter(paym, payx, dstm, zeros_nf)

    node2, xe2 = _run_final(sums, h2, att2, of2, cmix2, wn1_2, bn1_2,
                            lngn2, lnbn2, wn2_2, bn2_2)

    return (xe2[:N, 0:3], node2[:N], xe2[N:, 0:3], node2[N:])
